# baseline (speedup 1.0000x reference)
import jax
import jax.numpy as jnp
from jax import lax
from jax.experimental import pallas as pl
from jax.experimental.pallas import tpu as pltpu
from jax.experimental.pallas import tpu_sc as plsc

N = 10000
E = 320000
D = 128
H = 64
G = 64
GF = 16

NC = 2
NS = 16
NW = NC * NS
KA = 128
NCA = 80
EP = NW * NCA * KA
NBLK = EP // KA
ACCN = 10016
RPA = ACCN // NS
JUNK = N
NPAD = 10240
RPT = NPAD // NS

_mesh = plsc.VectorSubcoreMesh(
    core_axis_name="c", subcore_axis_name="s", num_cores=NC, num_subcores=NS)


def _deg_body(eidx_hbm, out_hbm, idxv, ones_v, zbuf, acc, sem):
    c = lax.axis_index("c")
    s = lax.axis_index("s")
    w = c * NS + s
    pltpu.sync_copy(eidx_hbm.at[pl.ds(w * NCA, NCA)], idxv)
    for i in range(8):
        ones_v[pl.ds(i * 16, 16)] = jnp.ones((16,), jnp.float32)
        zbuf[pl.ds(i * 16, 16)] = jnp.zeros((16,), jnp.float32)
    r0 = s * RPT
    for i in range(RPT // 128):
        pltpu.sync_copy(zbuf, acc.at[pl.ds(r0 + i * 128, 128)])
    plsc.subcore_barrier()

    def fire(j, carry):
        pltpu.async_copy(ones_v, acc.at[idxv.at[j, 1]], sem, add=True)
        return carry

    lax.fori_loop(0, NCA, fire, 0)

    def drain(j, carry):
        pltpu.make_async_copy(ones_v, acc.at[idxv.at[j, 1]], sem).wait()
        return carry

    lax.fori_loop(0, NCA, drain, 0)
    plsc.subcore_barrier()
    pltpu.sync_copy(acc.at[pl.ds(r0, RPT)], out_hbm.at[c, pl.ds(r0, RPT)])


_deg_kernel = pl.kernel(
    _deg_body,
    out_type=jax.ShapeDtypeStruct((NC, NPAD), jnp.float32),
    mesh=_mesh,
    scratch_types=[
        pltpu.VMEM((NCA, 2, KA), jnp.int32),
        pltpu.VMEM((KA,), jnp.float32),
        pltpu.VMEM((128,), jnp.float32),
        pltpu.VMEM_SHARED((NPAD,), jnp.float32),
        pltpu.SemaphoreType.DMA,
    ],
)


def _agg_body(g_hbm, eidx_hbm, out_hbm,
              idxv, b0, b1, b2, b3, b4, b5, b6, b7,
              g0, g1, g2, g3, g4, g5, g6, g7,
              s0, s1, s2, s3, s4, s5, s6, s7, acc):
    bufs = [b0, b1, b2, b3, b4, b5, b6, b7]
    gs = [g0, g1, g2, g3, g4, g5, g6, g7]
    ss = [s0, s1, s2, s3, s4, s5, s6, s7]
    c = lax.axis_index("c")
    s = lax.axis_index("s")
    w = c * NS + s
    pltpu.sync_copy(eidx_hbm.at[pl.ds(w * NCA, NCA)], idxv)

    def zrow(i, carry):
        for kk in range(H // 16):
            b0[i, pl.ds(16 * kk, 16)] = jnp.zeros((16,), jnp.float32)
        return carry

    lax.fori_loop(0, KA, zrow, 0)
    r0 = s * RPA

    @pl.when(jnp.logical_and(c == 0, s < NS - 1))
    def _():
        pltpu.sync_copy(g_hbm.at[pl.ds(r0, RPA)], acc.at[pl.ds(r0, RPA)])

    @pl.when(jnp.logical_and(c == 0, s == NS - 1))
    def _():
        pltpu.sync_copy(g_hbm.at[pl.ds((NS - 1) * RPA, N - (NS - 1) * RPA)],
                        acc.at[pl.ds((NS - 1) * RPA, N - (NS - 1) * RPA)])
        pltpu.sync_copy(b0.at[pl.ds(0, ACCN - N)], acc.at[pl.ds(N, ACCN - N)])

    @pl.when(c == 1)
    def _():
        for i in range(RPA // KA):
            pltpu.sync_copy(b0, acc.at[pl.ds(r0 + i * KA, KA)])
        pltpu.sync_copy(b0.at[pl.ds(0, RPA % KA)],
                        acc.at[pl.ds(r0 + (RPA // KA) * KA, RPA % KA)])

    plsc.subcore_barrier()

    def gf(j, buf, sem):
        pltpu.async_copy(g_hbm.at[idxv.at[j, 0]], buf, sem)

    def gw(j, buf, sem):
        pltpu.make_async_copy(g_hbm.at[idxv.at[j, 0]], buf, sem).wait()

    def sf(j, buf, sem):
        pltpu.async_copy(buf, acc.at[idxv.at[j, 1]], sem, add=True)

    def sw(j, buf, sem):
        pltpu.make_async_copy(buf, acc.at[idxv.at[j, 1]], sem).wait()

    gf(0, bufs[0], gs[0])
    gf(1, bufs[1], gs[1])
    gf(2, bufs[2], gs[2])
    gf(3, bufs[3], gs[3])
    gf(4, bufs[4], gs[4])
    gf(5, bufs[5], gs[5])

    def step(t, carry):
        for k in range(4):
            j = 8 * t + 2 * k
            a0, a1 = 2 * k, 2 * k + 1
            p0, p1 = (2 * k - 2) % 8, (2 * k - 1) % 8
            gw(j, bufs[a0], gs[a0])
            gw(j + 1, bufs[a1], gs[a1])
            sf(j, bufs[a0], ss[a0])
            sf(j + 1, bufs[a1], ss[a1])

            def waits(j=j, p0=p0, p1=p1):
                sw(j - 2, bufs[p0], ss[p0])
                sw(j - 1, bufs[p1], ss[p1])

            def fires(j=j, p0=p0, p1=p1):
                gf(j + 6, bufs[p0], gs[p0])
                gf(j + 7, bufs[p1], gs[p1])

            if k == 0:
                pl.when(t > 0)(waits)
                fires()
            else:
                waits()
                pl.when(t < NCA // 8 - 1)(fires)
        return carry

    lax.fori_loop(0, NCA // 8, step, 0)
    sw(NCA - 2, bufs[6], ss[6])
    sw(NCA - 1, bufs[7], ss[7])
    plsc.subcore_barrier()
    pltpu.sync_copy(acc.at[pl.ds(r0, RPA)], out_hbm.at[c, pl.ds(r0, RPA)])


_agg_kernel = pl.kernel(
    _agg_body,
    out_type=jax.ShapeDtypeStruct((NC, ACCN, H), jnp.float32),
    mesh=_mesh,
    compiler_params=pltpu.CompilerParams(use_tc_tiling_on_sc=False),
    scratch_types=(
        [pltpu.VMEM((NCA, 2, KA), jnp.int32)]
        + [pltpu.VMEM((KA, H), jnp.float32)] * 8
        + [pltpu.SemaphoreType.DMA] * 16
        + [pltpu.VMEM_SHARED((ACCN, H), jnp.float32)]
    ),
)


def _tc1_body(x_ref, w1_ref, da_ref, db_ref, g1_ref, dinv_ref):
    dinv = lax.rsqrt(da_ref[...] + db_ref[...] + 1.0)
    h1 = jnp.dot(x_ref[...], w1_ref[...], preferred_element_type=jnp.float32)
    g1_ref[...] = h1 * dinv
    dinv_ref[...] = dinv


def _tc2_body(agg_ref, dinv_ref, b1_ref, w2_ref, g2_ref):
    t = (agg_ref[0, :N] + agg_ref[1, :N]) * dinv_ref[...] + b1_ref[...]
    t = jnp.maximum(t, 0.0)
    h2 = jnp.dot(t, w2_ref[...], preferred_element_type=jnp.float32)
    g2_ref[...] = h2 * dinv_ref[...]


def _tc3_body(agg_ref, dinv_ref, b2_ref, batch_ref, gat_ref,
              wp_ref, wg_ref, bf1_ref, wf2_ref, bf2_ref, out_ref):
    h = (agg_ref[0, :N] + agg_ref[1, :N]) * dinv_ref[...] + b2_ref[...]
    h = jnp.maximum(h, 0.0)
    gid = lax.broadcasted_iota(jnp.int32, (G, N), 0)
    mask = (gid == batch_ref[...]).astype(jnp.float32)
    counts = jnp.sum(mask, axis=1, keepdims=True)
    pooled = jnp.dot(mask, h, preferred_element_type=jnp.float32)
    pooled = pooled / jnp.maximum(counts, 1.0)
    z = (jnp.dot(pooled, wp_ref[...], preferred_element_type=jnp.float32)
         + jnp.dot(gat_ref[...], wg_ref[...], preferred_element_type=jnp.float32)
         + bf1_ref[...])
    z = jnp.maximum(z, 0.0)
    out_ref[...] = (jnp.dot(z, wf2_ref[...], preferred_element_type=jnp.float32)
                    + bf2_ref[...])


def _tc_call(body, out_shape, *args):
    return pl.pallas_call(body, out_shape=out_shape)(*args)


def kernel(x, edge_index, batch, global_attr, W1, b1, W2, b2,
           Wfc1, bfc1, Wfc2, bfc2):
    pad = jnp.stack([jnp.zeros((EP - E,), jnp.int32),
                     jnp.full((EP - E,), JUNK, jnp.int32)])
    ei = jnp.concatenate([edge_index, pad], axis=1)
    eidx = ei.reshape(2, NBLK, KA).transpose(1, 0, 2)

    degp = _deg_kernel(eidx)
    da = degp[0, :N].reshape(N, 1)
    db = degp[1, :N].reshape(N, 1)

    g1, dinv = _tc_call(
        _tc1_body,
        (jax.ShapeDtypeStruct((N, H), jnp.float32),
         jax.ShapeDtypeStruct((N, 1), jnp.float32)),
        x, W1, da, db)

    agg1 = _agg_kernel(g1, eidx)
    g2 = _tc_call(
        _tc2_body, jax.ShapeDtypeStruct((N, H), jnp.float32),
        agg1, dinv, b1.reshape(1, H), W2)

    agg2 = _agg_kernel(g2, eidx)
    out = _tc_call(
        _tc3_body, jax.ShapeDtypeStruct((G, 1), jnp.float32),
        agg2, dinv, b2.reshape(1, H),
        batch.reshape(1, N), global_attr,
        Wfc1[:H], Wfc1[H:], bfc1.reshape(1, 64), Wfc2, bfc2.reshape(1, 1))
    return out.reshape(G)


# speedup vs baseline: 1.0093x; 1.0093x over previous
import jax
import jax.numpy as jnp
from jax import lax
from jax.experimental import pallas as pl
from jax.experimental.pallas import tpu as pltpu
from jax.experimental.pallas import tpu_sc as plsc

N = 10000
E = 320000
D = 128
H = 64
G = 64
GF = 16

NC = 2
NS = 16
NW = NC * NS
KA = 128
NCA = 80
EP = NW * NCA * KA
NBLK = EP // KA
ACCN = 10240
RPA = ACCN // NS
NJUNK = ACCN - N
NPAD = 10240
RPT = NPAD // NS

_mesh = plsc.VectorSubcoreMesh(
    core_axis_name="c", subcore_axis_name="s", num_cores=NC, num_subcores=NS)


def _deg_body(eidx_hbm, out_hbm, idxv, ones_v, zbuf, acc, sem):
    c = lax.axis_index("c")
    s = lax.axis_index("s")
    w = c * NS + s
    pltpu.sync_copy(eidx_hbm.at[pl.ds(w * NCA, NCA)], idxv)
    for i in range(8):
        ones_v[pl.ds(i * 16, 16)] = jnp.ones((16,), jnp.float32)
        zbuf[pl.ds(i * 16, 16)] = jnp.zeros((16,), jnp.float32)
    r0 = s * RPT
    for i in range(RPT // 128):
        pltpu.sync_copy(zbuf, acc.at[pl.ds(r0 + i * 128, 128)])
    plsc.subcore_barrier()

    def fire(j, carry):
        pltpu.async_copy(ones_v, acc.at[idxv.at[j, 1]], sem, add=True)
        return carry

    lax.fori_loop(0, NCA, fire, 0)

    def drain(j, carry):
        pltpu.make_async_copy(ones_v, acc.at[idxv.at[j, 1]], sem).wait()
        return carry

    lax.fori_loop(0, NCA, drain, 0)
    plsc.subcore_barrier()
    pltpu.sync_copy(acc.at[pl.ds(r0, RPT)], out_hbm.at[c, pl.ds(r0, RPT)])


_deg_kernel = pl.kernel(
    _deg_body,
    out_type=jax.ShapeDtypeStruct((NC, NPAD), jnp.float32),
    mesh=_mesh,
    scratch_types=[
        pltpu.VMEM((NCA, 2, KA), jnp.int32),
        pltpu.VMEM((KA,), jnp.float32),
        pltpu.VMEM((128,), jnp.float32),
        pltpu.VMEM_SHARED((NPAD,), jnp.float32),
        pltpu.SemaphoreType.DMA,
    ],
)


def _agg_body(g_hbm, eidx_hbm, out_hbm,
              idxv, b0, b1, b2, b3, b4, b5, b6, b7,
              g0, g1, g2, g3, g4, g5, g6, g7,
              s0, s1, s2, s3, s4, s5, s6, s7, acc):
    bufs = [b0, b1, b2, b3, b4, b5, b6, b7]
    gs = [g0, g1, g2, g3, g4, g5, g6, g7]
    ss = [s0, s1, s2, s3, s4, s5, s6, s7]
    c = lax.axis_index("c")
    s = lax.axis_index("s")
    w = c * NS + s
    pltpu.sync_copy(eidx_hbm.at[pl.ds(w * NCA, NCA)], idxv)

    def zrow(i, carry):
        for kk in range(H // 16):
            b0[i, pl.ds(16 * kk, 16)] = jnp.zeros((16,), jnp.float32)
        return carry

    lax.fori_loop(0, KA, zrow, 0)
    r0 = s * RPA

    @pl.when(jnp.logical_and(c == 0, s < NS - 1))
    def _():
        pltpu.sync_copy(g_hbm.at[pl.ds(r0, RPA)], acc.at[pl.ds(r0, RPA)])

    @pl.when(jnp.logical_and(c == 0, s == NS - 1))
    def _():
        pltpu.sync_copy(g_hbm.at[pl.ds((NS - 1) * RPA, N - (NS - 1) * RPA)],
                        acc.at[pl.ds((NS - 1) * RPA, N - (NS - 1) * RPA)])
        for i in range(NJUNK // 128):
            pltpu.sync_copy(b0, acc.at[pl.ds(N + i * 128, 128)])
        pltpu.sync_copy(b0.at[pl.ds(0, NJUNK % 128)],
                        acc.at[pl.ds(N + (NJUNK // 128) * 128, NJUNK % 128)])

    @pl.when(c == 1)
    def _():
        for i in range(RPA // KA):
            pltpu.sync_copy(b0, acc.at[pl.ds(r0 + i * KA, KA)])

    plsc.subcore_barrier()

    def gf(j, buf, sem):
        pltpu.async_copy(g_hbm.at[idxv.at[j, 0]], buf, sem)

    def gw(j, buf, sem):
        pltpu.make_async_copy(g_hbm.at[idxv.at[j, 0]], buf, sem).wait()

    def sf(j, buf, sem):
        pltpu.async_copy(buf, acc.at[idxv.at[j, 1]], sem, add=True)

    def sw(j, buf, sem):
        pltpu.make_async_copy(buf, acc.at[idxv.at[j, 1]], sem).wait()

    gf(0, bufs[0], gs[0])
    gf(1, bufs[1], gs[1])
    gf(2, bufs[2], gs[2])
    gf(3, bufs[3], gs[3])
    gf(4, bufs[4], gs[4])
    gf(5, bufs[5], gs[5])

    def step(t, carry):
        for k in range(4):
            j = 8 * t + 2 * k
            a0, a1 = 2 * k, 2 * k + 1
            p0, p1 = (2 * k - 2) % 8, (2 * k - 1) % 8
            gw(j, bufs[a0], gs[a0])
            gw(j + 1, bufs[a1], gs[a1])
            sf(j, bufs[a0], ss[a0])
            sf(j + 1, bufs[a1], ss[a1])

            def waits(j=j, p0=p0, p1=p1):
                sw(j - 2, bufs[p0], ss[p0])
                sw(j - 1, bufs[p1], ss[p1])

            def fires(j=j, p0=p0, p1=p1):
                gf(j + 6, bufs[p0], gs[p0])
                gf(j + 7, bufs[p1], gs[p1])

            if k == 0:
                pl.when(t > 0)(waits)
                fires()
            else:
                waits()
                pl.when(t < NCA // 8 - 1)(fires)
        return carry

    lax.fori_loop(0, NCA // 8, step, 0)
    sw(NCA - 2, bufs[6], ss[6])
    sw(NCA - 1, bufs[7], ss[7])
    plsc.subcore_barrier()
    pltpu.sync_copy(acc.at[pl.ds(r0, RPA)], out_hbm.at[c, pl.ds(r0, RPA)])


_agg_kernel = pl.kernel(
    _agg_body,
    out_type=jax.ShapeDtypeStruct((NC, ACCN, H), jnp.float32),
    mesh=_mesh,
    compiler_params=pltpu.CompilerParams(use_tc_tiling_on_sc=False),
    scratch_types=(
        [pltpu.VMEM((NCA, 2, KA), jnp.int32)]
        + [pltpu.VMEM((KA, H), jnp.float32)] * 8
        + [pltpu.SemaphoreType.DMA] * 16
        + [pltpu.VMEM_SHARED((ACCN, H), jnp.float32)]
    ),
)


def _tc1_body(x_ref, w1_ref, da_ref, db_ref, g1_ref, dinv_ref):
    dinv = lax.rsqrt(da_ref[...] + db_ref[...] + 1.0)
    h1 = jnp.dot(x_ref[...], w1_ref[...], preferred_element_type=jnp.float32)
    g1_ref[...] = h1 * dinv
    dinv_ref[...] = dinv


def _tc2_body(agg_ref, dinv_ref, b1_ref, w2_ref, g2_ref):
    t = (agg_ref[0, :N] + agg_ref[1, :N]) * dinv_ref[...] + b1_ref[...]
    t = jnp.maximum(t, 0.0)
    h2 = jnp.dot(t, w2_ref[...], preferred_element_type=jnp.float32)
    g2_ref[...] = h2 * dinv_ref[...]


def _tc3_body(agg_ref, dinv_ref, b2_ref, batch_ref, gat_ref,
              wp_ref, wg_ref, bf1_ref, wf2_ref, bf2_ref, out_ref):
    h = (agg_ref[0, :N] + agg_ref[1, :N]) * dinv_ref[...] + b2_ref[...]
    h = jnp.maximum(h, 0.0)
    gid = lax.broadcasted_iota(jnp.int32, (G, N), 0)
    mask = (gid == batch_ref[...]).astype(jnp.float32)
    counts = jnp.sum(mask, axis=1, keepdims=True)
    pooled = jnp.dot(mask, h, preferred_element_type=jnp.float32)
    pooled = pooled / jnp.maximum(counts, 1.0)
    z = (jnp.dot(pooled, wp_ref[...], preferred_element_type=jnp.float32)
         + jnp.dot(gat_ref[...], wg_ref[...], preferred_element_type=jnp.float32)
         + bf1_ref[...])
    z = jnp.maximum(z, 0.0)
    out_ref[...] = (jnp.dot(z, wf2_ref[...], preferred_element_type=jnp.float32)
                    + bf2_ref[...])


def _tc_call(body, out_shape, *args):
    return pl.pallas_call(body, out_shape=out_shape)(*args)


def kernel(x, edge_index, batch, global_attr, W1, b1, W2, b2,
           Wfc1, bfc1, Wfc2, bfc2):
    pad = jnp.stack([jnp.zeros((EP - E,), jnp.int32),
                     N + jnp.arange(EP - E, dtype=jnp.int32) % NJUNK])
    ei = jnp.concatenate([edge_index, pad], axis=1)
    eidx = ei.reshape(2, NBLK, KA).transpose(1, 0, 2)

    degp = _deg_kernel(eidx)
    da = degp[0, :N].reshape(N, 1)
    db = degp[1, :N].reshape(N, 1)

    g1, dinv = _tc_call(
        _tc1_body,
        (jax.ShapeDtypeStruct((N, H), jnp.float32),
         jax.ShapeDtypeStruct((N, 1), jnp.float32)),
        x, W1, da, db)

    agg1 = _agg_kernel(g1, eidx)
    g2 = _tc_call(
        _tc2_body, jax.ShapeDtypeStruct((N, H), jnp.float32),
        agg1, dinv, b1.reshape(1, H), W2)

    agg2 = _agg_kernel(g2, eidx)
    out = _tc_call(
        _tc3_body, jax.ShapeDtypeStruct((G, 1), jnp.float32),
        agg2, dinv, b2.reshape(1, H),
        batch.reshape(1, N), global_attr,
        Wfc1[:H], Wfc1[H:], bfc1.reshape(1, 64), Wfc2, bfc2.reshape(1, 1))
    return out.reshape(G)


# speedup vs baseline: 2.7370x; 2.7118x over previous
import jax
import jax.numpy as jnp
from jax import lax
from jax.experimental import pallas as pl
from jax.experimental.pallas import tpu as pltpu
from jax.experimental.pallas import tpu_sc as plsc

N = 10000
E = 320000
D = 128
H = 64
G = 64
GF = 16

NC = 2
NS = 16
NW = NC * NS
KA = 128
NCA = 80
EP = NW * NCA * KA
NBLK = EP // KA
ACCN = 10240
RPA = ACCN // NS
NJUNK = ACCN - N
NPAD = 10240
RPT = NPAD // NS

_mesh = plsc.VectorSubcoreMesh(
    core_axis_name="c", subcore_axis_name="s", num_cores=NC, num_subcores=NS)


def _deg_body(eidx_hbm, out_hbm, idxv, ones_v, zbuf, acc, sem):
    c = lax.axis_index("c")
    s = lax.axis_index("s")
    w = c * NS + s
    pltpu.sync_copy(eidx_hbm.at[pl.ds(w * NCA, NCA)], idxv)
    for i in range(8):
        ones_v[pl.ds(i * 16, 16)] = jnp.ones((16,), jnp.float32)
        zbuf[pl.ds(i * 16, 16)] = jnp.zeros((16,), jnp.float32)
    r0 = s * RPT
    for i in range(RPT // 128):
        pltpu.sync_copy(zbuf, acc.at[pl.ds(r0 + i * 128, 128)])
    plsc.subcore_barrier()

    def fire(j, carry):
        pltpu.async_copy(ones_v, acc.at[idxv.at[j, 1]], sem, add=True)
        return carry

    lax.fori_loop(0, NCA, fire, 0)

    def drain(j, carry):
        pltpu.make_async_copy(ones_v, acc.at[idxv.at[j, 1]], sem).wait()
        return carry

    lax.fori_loop(0, NCA, drain, 0)
    plsc.subcore_barrier()
    pltpu.sync_copy(acc.at[pl.ds(r0, RPT)], out_hbm.at[c, pl.ds(r0, RPT)])


_deg_kernel = pl.kernel(
    _deg_body,
    out_type=jax.ShapeDtypeStruct((NC, NPAD), jnp.float32),
    mesh=_mesh,
    scratch_types=[
        pltpu.VMEM((NCA, 2, KA), jnp.int32),
        pltpu.VMEM((KA,), jnp.float32),
        pltpu.VMEM((128,), jnp.float32),
        pltpu.VMEM_SHARED((NPAD,), jnp.float32),
        pltpu.SemaphoreType.DMA,
    ],
)


def _agg_body(g_hbm, eidx_hbm, out_hbm,
              idxv, b0, b1, b2, b3, b4, b5, b6, b7,
              g0, g1, g2, g3, g4, g5, g6, g7,
              s0, s1, s2, s3, s4, s5, s6, s7, acc):
    bufs = [b0, b1, b2, b3, b4, b5, b6, b7]
    gs = [g0, g1, g2, g3, g4, g5, g6, g7]
    ss = [s0, s1, s2, s3, s4, s5, s6, s7]
    c = lax.axis_index("c")
    s = lax.axis_index("s")
    w = c * NS + s
    pltpu.sync_copy(eidx_hbm.at[pl.ds(w * NCA, NCA)], idxv)

    def zrow(i, carry):
        for kk in range(H // 16):
            b0[i, pl.ds(16 * kk, 16)] = jnp.zeros((16,), jnp.float32)
        return carry

    lax.fori_loop(0, KA, zrow, 0)
    r0 = s * RPA

    @pl.when(jnp.logical_and(c == 0, s < NS - 1))
    def _():
        pltpu.sync_copy(g_hbm.at[pl.ds(r0, RPA)], acc.at[pl.ds(r0, RPA)])

    @pl.when(jnp.logical_and(c == 0, s == NS - 1))
    def _():
        pltpu.sync_copy(g_hbm.at[pl.ds((NS - 1) * RPA, N - (NS - 1) * RPA)],
                        acc.at[pl.ds((NS - 1) * RPA, N - (NS - 1) * RPA)])
        for i in range(NJUNK // 128):
            pltpu.sync_copy(b0, acc.at[pl.ds(N + i * 128, 128)])
        pltpu.sync_copy(b0.at[pl.ds(0, NJUNK % 128)],
                        acc.at[pl.ds(N + (NJUNK // 128) * 128, NJUNK % 128)])

    @pl.when(c == 1)
    def _():
        for i in range(RPA // KA):
            pltpu.sync_copy(b0, acc.at[pl.ds(r0 + i * KA, KA)])

    plsc.subcore_barrier()

    def gf(j, buf, sem):
        pltpu.async_copy(g_hbm.at[idxv.at[j, 0]], buf, sem)

    def gw(j, buf, sem):
        pltpu.make_async_copy(g_hbm.at[idxv.at[j, 0]], buf, sem).wait()

    def sf(j, buf, sem):
        pltpu.async_copy(buf, acc.at[idxv.at[j, 1]], sem, add=True)

    def sw(j, buf, sem):
        pltpu.make_async_copy(buf, acc.at[idxv.at[j, 1]], sem).wait()

    gf(0, bufs[0], gs[0])
    gf(1, bufs[1], gs[1])
    gf(2, bufs[2], gs[2])
    gf(3, bufs[3], gs[3])
    gf(4, bufs[4], gs[4])
    gf(5, bufs[5], gs[5])

    def step(t, carry):
        for k in range(4):
            j = 8 * t + 2 * k
            a0, a1 = 2 * k, 2 * k + 1
            p0, p1 = (2 * k - 2) % 8, (2 * k - 1) % 8
            gw(j, bufs[a0], gs[a0])
            gw(j + 1, bufs[a1], gs[a1])
            sf(j, bufs[a0], ss[a0])
            sf(j + 1, bufs[a1], ss[a1])

            def waits(j=j, p0=p0, p1=p1):
                sw(j - 2, bufs[p0], ss[p0])
                sw(j - 1, bufs[p1], ss[p1])

            def fires(j=j, p0=p0, p1=p1):
                gf(j + 6, bufs[p0], gs[p0])
                gf(j + 7, bufs[p1], gs[p1])

            if k == 0:
                pl.when(t > 0)(waits)
                fires()
            else:
                waits()
                pl.when(t < NCA // 8 - 1)(fires)
        return carry

    lax.fori_loop(0, NCA // 8, step, 0)
    sw(NCA - 2, bufs[6], ss[6])
    sw(NCA - 1, bufs[7], ss[7])
    plsc.subcore_barrier()
    pltpu.sync_copy(acc.at[pl.ds(r0, RPA)], out_hbm.at[c, pl.ds(r0, RPA)])


_agg_kernel = pl.kernel(
    _agg_body,
    out_type=jax.ShapeDtypeStruct((NC, ACCN, H), jnp.float32),
    mesh=_mesh,
    compiler_params=pltpu.CompilerParams(use_tc_tiling_on_sc=False),
    scratch_types=(
        [pltpu.VMEM((NCA, 2, KA), jnp.int32)]
        + [pltpu.VMEM((KA, H), jnp.float32)] * 8
        + [pltpu.SemaphoreType.DMA] * 16
        + [pltpu.VMEM_SHARED((ACCN, H), jnp.float32)]
    ),
)


def _tc1_body(x_ref, w1_ref, da_ref, db_ref, g1_ref, dinv_ref):
    dinv = lax.rsqrt(da_ref[...] + db_ref[...] + 1.0)
    h1 = jnp.dot(x_ref[...], w1_ref[...], preferred_element_type=jnp.float32)
    g1_ref[...] = h1 * dinv
    dinv_ref[...] = dinv


def _tc2_body(agg_ref, dinv_ref, b1_ref, w2_ref, g2_ref):
    t = (agg_ref[0, :N] + agg_ref[1, :N]) * dinv_ref[...] + b1_ref[...]
    t = jnp.maximum(t, 0.0)
    h2 = jnp.dot(t, w2_ref[...], preferred_element_type=jnp.float32)
    g2_ref[...] = h2 * dinv_ref[...]


def _tc3_body(agg_ref, dinv_ref, b2_ref, batch_ref, gat_ref,
              wp_ref, wg_ref, bf1_ref, wf2_ref, bf2_ref, out_ref):
    h = (agg_ref[0, :N] + agg_ref[1, :N]) * dinv_ref[...] + b2_ref[...]
    h = jnp.maximum(h, 0.0)
    gid = lax.broadcasted_iota(jnp.int32, (G, N), 0)
    mask = (gid == batch_ref[...]).astype(jnp.float32)
    counts = jnp.sum(mask, axis=1, keepdims=True)
    pooled = jnp.dot(mask, h, preferred_element_type=jnp.float32)
    pooled = pooled / jnp.maximum(counts, 1.0)
    z = (jnp.dot(pooled, wp_ref[...], preferred_element_type=jnp.float32)
         + jnp.dot(gat_ref[...], wg_ref[...], preferred_element_type=jnp.float32)
         + bf1_ref[...])
    z = jnp.maximum(z, 0.0)
    out_ref[...] = (jnp.dot(z, wf2_ref[...], preferred_element_type=jnp.float32)
                    + bf2_ref[...])


def _tc_call(body, out_shape, *args):
    return pl.pallas_call(body, out_shape=out_shape)(*args)


def kernel(x, edge_index, batch, global_attr, W1, b1, W2, b2,
           Wfc1, bfc1, Wfc2, bfc2):
    pad_i = jnp.arange(EP - E, dtype=jnp.int32)
    pad = jnp.stack([pad_i * 13 % N, N + pad_i % NJUNK])
    ei = jnp.concatenate([edge_index, pad], axis=1)
    eidx = ei.reshape(2, NBLK, KA).transpose(1, 0, 2)

    degp = _deg_kernel(eidx)
    da = degp[0, :N].reshape(N, 1)
    db = degp[1, :N].reshape(N, 1)

    g1, dinv = _tc_call(
        _tc1_body,
        (jax.ShapeDtypeStruct((N, H), jnp.float32),
         jax.ShapeDtypeStruct((N, 1), jnp.float32)),
        x, W1, da, db)

    agg1 = _agg_kernel(g1, eidx)
    g2 = _tc_call(
        _tc2_body, jax.ShapeDtypeStruct((N, H), jnp.float32),
        agg1, dinv, b1.reshape(1, H), W2)

    agg2 = _agg_kernel(g2, eidx)
    out = _tc_call(
        _tc3_body, jax.ShapeDtypeStruct((G, 1), jnp.float32),
        agg2, dinv, b2.reshape(1, H),
        batch.reshape(1, N), global_attr,
        Wfc1[:H], Wfc1[H:], bfc1.reshape(1, 64), Wfc2, bfc2.reshape(1, 1))
    return out.reshape(G)


# speedup vs baseline: 3.0754x; 1.1237x over previous
import jax
import jax.numpy as jnp
from jax import lax
from jax.experimental import pallas as pl
from jax.experimental.pallas import tpu as pltpu
from jax.experimental.pallas import tpu_sc as plsc

N = 10000
E = 320000
D = 128
H = 64
G = 64
GF = 16

NC = 2
NS = 16
NW = NC * NS
KA = 128
NCA = 80
EP = NW * NCA * KA
NBLK = EP // KA
ACCN = 10240
RPA = ACCN // NS
NJUNK = ACCN - N
NPAD = 10240
RPT = NPAD // NS

_mesh = plsc.VectorSubcoreMesh(
    core_axis_name="c", subcore_axis_name="s", num_cores=NC, num_subcores=NS)


def _deg_body(eidx_hbm, out_hbm, idxv, ones_v, zbuf, dld, dbb, acc, sem):
    c = lax.axis_index("c")
    s = lax.axis_index("s")
    w = c * NS + s
    pltpu.sync_copy(eidx_hbm.at[pl.ds(w * NCA, NCA)], idxv)
    for i in range(8):
        ones_v[pl.ds(i * 16, 16)] = jnp.ones((16,), jnp.float32)
        zbuf[pl.ds(i * 16, 16)] = jnp.zeros((16,), jnp.float32)
    r0 = s * RPT
    for i in range(RPT // 128):
        pltpu.sync_copy(zbuf, acc.at[pl.ds(r0 + i * 128, 128)])
    plsc.subcore_barrier()

    def fire(j, carry):
        pltpu.async_copy(ones_v, acc.at[idxv.at[j, 1]], sem, add=True)
        return carry

    lax.fori_loop(0, NCA, fire, 0)

    def drain(j, carry):
        pltpu.make_async_copy(ones_v, acc.at[idxv.at[j, 1]], sem).wait()
        return carry

    lax.fori_loop(0, NCA, drain, 0)
    plsc.subcore_barrier()
    pltpu.sync_copy(acc.at[pl.ds(r0, RPT)], dld)

    def bcast(i, carry):
        v0 = plsc.load_gather(dld, [jnp.full((16,), 2 * i, jnp.int32)])
        v1 = plsc.load_gather(dld, [jnp.full((16,), 2 * i + 1, jnp.int32)])
        for k in range(4):
            dbb[i, pl.ds(16 * k, 16)] = v0
        for k in range(4, 8):
            dbb[i, pl.ds(16 * k, 16)] = v1
        return carry

    lax.fori_loop(0, RPT // 2, bcast, 0)
    pltpu.sync_copy(dbb, out_hbm.at[c, pl.ds(s * (RPT // 2), RPT // 2)])


_deg_kernel = pl.kernel(
    _deg_body,
    out_type=jax.ShapeDtypeStruct((NC, NPAD // 2, 128), jnp.float32),
    mesh=_mesh,
    compiler_params=pltpu.CompilerParams(needs_layout_passes=False),
    scratch_types=[
        pltpu.VMEM((NCA, 2, KA), jnp.int32),
        pltpu.VMEM((KA,), jnp.float32),
        pltpu.VMEM((128,), jnp.float32),
        pltpu.VMEM((RPT,), jnp.float32),
        pltpu.VMEM((RPT // 2, 128), jnp.float32),
        pltpu.VMEM_SHARED((NPAD,), jnp.float32),
        pltpu.SemaphoreType.DMA,
    ],
)


def _agg_body(g_hbm, eidx_hbm, out_hbm,
              idxv, b0, b1, b2, b3, b4, b5, b6, b7,
              g0, g1, g2, g3, g4, g5, g6, g7,
              s0, s1, s2, s3, s4, s5, s6, s7, acc):
    bufs = [b0, b1, b2, b3, b4, b5, b6, b7]
    gs = [g0, g1, g2, g3, g4, g5, g6, g7]
    ss = [s0, s1, s2, s3, s4, s5, s6, s7]
    c = lax.axis_index("c")
    s = lax.axis_index("s")
    w = c * NS + s
    pltpu.sync_copy(eidx_hbm.at[pl.ds(w * NCA, NCA)], idxv)

    def zrow(i, carry):
        for kk in range(H // 16):
            b0[i, pl.ds(16 * kk, 16)] = jnp.zeros((16,), jnp.float32)
        return carry

    lax.fori_loop(0, KA, zrow, 0)
    r0 = s * RPA

    @pl.when(jnp.logical_and(c == 0, s < NS - 1))
    def _():
        pltpu.sync_copy(g_hbm.at[pl.ds(r0, RPA)], acc.at[pl.ds(r0, RPA)])

    @pl.when(jnp.logical_and(c == 0, s == NS - 1))
    def _():
        pltpu.sync_copy(g_hbm.at[pl.ds((NS - 1) * RPA, N - (NS - 1) * RPA)],
                        acc.at[pl.ds((NS - 1) * RPA, N - (NS - 1) * RPA)])
        for i in range(NJUNK // 128):
            pltpu.sync_copy(b0, acc.at[pl.ds(N + i * 128, 128)])
        pltpu.sync_copy(b0.at[pl.ds(0, NJUNK % 128)],
                        acc.at[pl.ds(N + (NJUNK // 128) * 128, NJUNK % 128)])

    @pl.when(c == 1)
    def _():
        for i in range(RPA // KA):
            pltpu.sync_copy(b0, acc.at[pl.ds(r0 + i * KA, KA)])

    plsc.subcore_barrier()

    def gf(j, buf, sem):
        pltpu.async_copy(g_hbm.at[idxv.at[j, 0]], buf, sem)

    def gw(j, buf, sem):
        pltpu.make_async_copy(g_hbm.at[idxv.at[j, 0]], buf, sem).wait()

    def sf(j, buf, sem):
        pltpu.async_copy(buf, acc.at[idxv.at[j, 1]], sem, add=True)

    def sw(j, buf, sem):
        pltpu.make_async_copy(buf, acc.at[idxv.at[j, 1]], sem).wait()

    gf(0, bufs[0], gs[0])
    gf(1, bufs[1], gs[1])
    gf(2, bufs[2], gs[2])
    gf(3, bufs[3], gs[3])
    gf(4, bufs[4], gs[4])
    gf(5, bufs[5], gs[5])

    def step(t, carry):
        for k in range(4):
            j = 8 * t + 2 * k
            a0, a1 = 2 * k, 2 * k + 1
            p0, p1 = (2 * k - 2) % 8, (2 * k - 1) % 8
            gw(j, bufs[a0], gs[a0])
            gw(j + 1, bufs[a1], gs[a1])
            sf(j, bufs[a0], ss[a0])
            sf(j + 1, bufs[a1], ss[a1])

            def waits(j=j, p0=p0, p1=p1):
                sw(j - 2, bufs[p0], ss[p0])
                sw(j - 1, bufs[p1], ss[p1])

            def fires(j=j, p0=p0, p1=p1):
                gf(j + 6, bufs[p0], gs[p0])
                gf(j + 7, bufs[p1], gs[p1])

            if k == 0:
                pl.when(t > 0)(waits)
                fires()
            else:
                waits()
                pl.when(t < NCA // 8 - 1)(fires)
        return carry

    lax.fori_loop(0, NCA // 8, step, 0)
    sw(NCA - 2, bufs[6], ss[6])
    sw(NCA - 1, bufs[7], ss[7])
    plsc.subcore_barrier()
    pltpu.sync_copy(acc.at[pl.ds(r0, RPA)], out_hbm.at[c, pl.ds(r0, RPA)])


_agg_kernel = pl.kernel(
    _agg_body,
    out_type=jax.ShapeDtypeStruct((NC, ACCN, H), jnp.float32),
    mesh=_mesh,
    compiler_params=pltpu.CompilerParams(use_tc_tiling_on_sc=False),
    scratch_types=(
        [pltpu.VMEM((NCA, 2, KA), jnp.int32)]
        + [pltpu.VMEM((KA, H), jnp.float32)] * 8
        + [pltpu.SemaphoreType.DMA] * 16
        + [pltpu.VMEM_SHARED((ACCN, H), jnp.float32)]
    ),
)


NPR = N // 2
BROW = NC * ACCN * H // 128
BOFF = ACCN * H // 128


def _tc1_body(xe_ref, xo_ref, w1_ref, degb_ref, g1_ref, dinv_ref):
    dsum = degb_ref[0, :NPR] + degb_ref[1, :NPR]
    dinv = lax.rsqrt(dsum + 1.0)
    he = jnp.dot(xe_ref[...], w1_ref[...], preferred_element_type=jnp.float32)
    ho = jnp.dot(xo_ref[...], w1_ref[...], preferred_element_type=jnp.float32)
    h1p = jnp.concatenate([he, ho], axis=1)
    g1_ref[...] = h1p * dinv
    dinv_ref[...] = dinv


def _tc2_body(aggv_ref, dinv_ref, b1_ref, w2bd_ref, g2_ref):
    t = ((aggv_ref[0:NPR] + aggv_ref[BOFF:BOFF + NPR]) * dinv_ref[...]
         + b1_ref[...])
    t = jnp.maximum(t, 0.0)
    h2p = jnp.dot(t, w2bd_ref[...], preferred_element_type=jnp.float32)
    g2_ref[...] = h2p * dinv_ref[...]


def _tc3_body(aggv_ref, dinv_ref, b2_ref, bate_ref, bato_ref, gat_ref,
              wp_ref, wg_ref, bf1_ref, wf2_ref, bf2_ref, out_ref):
    h = ((aggv_ref[0:NPR] + aggv_ref[BOFF:BOFF + NPR]) * dinv_ref[...]
         + b2_ref[...])
    h = jnp.maximum(h, 0.0)
    gid = lax.broadcasted_iota(jnp.int32, (G, NPR), 0)
    maske = (gid == bate_ref[...]).astype(jnp.float32)
    masko = (gid == bato_ref[...]).astype(jnp.float32)
    counts = (jnp.sum(maske, axis=1, keepdims=True)
              + jnp.sum(masko, axis=1, keepdims=True))
    pe = jnp.dot(maske, h, preferred_element_type=jnp.float32)
    po = jnp.dot(masko, h, preferred_element_type=jnp.float32)
    pooled = pe[:, 0:H] + po[:, H:2 * H]
    pooled = pooled / jnp.maximum(counts, 1.0)
    z = (jnp.dot(pooled, wp_ref[...], preferred_element_type=jnp.float32)
         + jnp.dot(gat_ref[...], wg_ref[...], preferred_element_type=jnp.float32)
         + bf1_ref[...])
    z = jnp.maximum(z, 0.0)
    out_ref[...] = (jnp.dot(z, wf2_ref[...], preferred_element_type=jnp.float32)
                    + bf2_ref[...])


def _tc_call(body, out_shape, *args):
    return pl.pallas_call(body, out_shape=out_shape)(*args)


def kernel(x, edge_index, batch, global_attr, W1, b1, W2, b2,
           Wfc1, bfc1, Wfc2, bfc2):
    pad_i = jnp.arange(EP - E, dtype=jnp.int32)
    pad = jnp.stack([pad_i * 13 % N, N + pad_i % NJUNK])
    ei = jnp.concatenate([edge_index, pad], axis=1)
    eidx = ei.reshape(2, NBLK, KA).transpose(1, 0, 2)

    degb = _deg_kernel(eidx)

    xe = x[0::2]
    xo = x[1::2]
    g1p, dinvp = _tc_call(
        _tc1_body,
        (jax.ShapeDtypeStruct((NPR, 2 * H), jnp.float32),
         jax.ShapeDtypeStruct((NPR, 2 * H), jnp.float32)),
        xe, xo, W1, degb)

    zb = jnp.zeros((H, H), jnp.float32)
    w2bd = jnp.block([[W2, zb], [zb, W2]])
    b1p = jnp.concatenate([b1, b1]).reshape(1, 2 * H)
    b2p = jnp.concatenate([b2, b2]).reshape(1, 2 * H)
    bpair = batch.reshape(NPR, 2)
    bate = bpair[:, 0].reshape(1, NPR)
    bato = bpair[:, 1].reshape(1, NPR)

    agg1 = _agg_kernel(g1p.reshape(N, H), eidx)
    g2p = _tc_call(
        _tc2_body, jax.ShapeDtypeStruct((NPR, 2 * H), jnp.float32),
        agg1.reshape(BROW, 128), dinvp, b1p, w2bd)

    agg2 = _agg_kernel(g2p.reshape(N, H), eidx)
    out = _tc_call(
        _tc3_body, jax.ShapeDtypeStruct((G, 1), jnp.float32),
        agg2.reshape(BROW, 128), dinvp, b2p, bate, bato, global_attr,
        Wfc1[:H], Wfc1[H:], bfc1.reshape(1, 64), Wfc2, bfc2.reshape(1, 1))
    return out.reshape(G)


# speedup vs baseline: 3.4142x; 1.1101x over previous
import jax
import jax.numpy as jnp
from jax import lax
from jax.experimental import pallas as pl
from jax.experimental.pallas import tpu as pltpu
from jax.experimental.pallas import tpu_sc as plsc

N = 10000
E = 320000
D = 128
H = 64
G = 64
GF = 16

NC = 2
NS = 16
NW = NC * NS
KA = 128
NCA = 80
EP = NW * NCA * KA
NBLK = EP // KA
ACCN = 10240
RPA = ACCN // NS
NJUNK = ACCN - N
NPAD = 10240
RPT = NPAD // NS

_mesh = plsc.VectorSubcoreMesh(
    core_axis_name="c", subcore_axis_name="s", num_cores=NC, num_subcores=NS)


def _deg_body(eidx_hbm, out_hbm, idxv, ones_v, zbuf, dld, dbb, acc, sem):
    c = lax.axis_index("c")
    s = lax.axis_index("s")
    w = c * NS + s
    pltpu.sync_copy(eidx_hbm.at[pl.ds(w * NCA, NCA)], idxv)
    for i in range(8):
        ones_v[pl.ds(i * 16, 16)] = jnp.ones((16,), jnp.float32)
        zbuf[pl.ds(i * 16, 16)] = jnp.zeros((16,), jnp.float32)
    r0 = s * RPT
    for i in range(RPT // 128):
        pltpu.sync_copy(zbuf, acc.at[pl.ds(r0 + i * 128, 128)])
    plsc.subcore_barrier()

    def fire(j, carry):
        pltpu.async_copy(ones_v, acc.at[idxv.at[j, 1]], sem, add=True)
        return carry

    lax.fori_loop(0, NCA, fire, 0)

    def drain(j, carry):
        pltpu.make_async_copy(ones_v, acc.at[idxv.at[j, 1]], sem).wait()
        return carry

    lax.fori_loop(0, NCA, drain, 0)
    plsc.subcore_barrier()
    pltpu.sync_copy(acc.at[pl.ds(r0, RPT)], dld)

    def bcast(i, carry):
        v0 = plsc.load_gather(dld, [jnp.full((16,), 2 * i, jnp.int32)])
        v1 = plsc.load_gather(dld, [jnp.full((16,), 2 * i + 1, jnp.int32)])
        for k in range(4):
            dbb[i, pl.ds(16 * k, 16)] = v0
        for k in range(4, 8):
            dbb[i, pl.ds(16 * k, 16)] = v1
        return carry

    lax.fori_loop(0, RPT // 2, bcast, 0)
    pltpu.sync_copy(dbb, out_hbm.at[c, pl.ds(s * (RPT // 2), RPT // 2)])


_deg_kernel = pl.kernel(
    _deg_body,
    out_type=jax.ShapeDtypeStruct((NC, NPAD // 2, 128), jnp.float32),
    mesh=_mesh,
    compiler_params=pltpu.CompilerParams(needs_layout_passes=False),
    scratch_types=[
        pltpu.VMEM((NCA, 2, KA), jnp.int32),
        pltpu.VMEM((KA,), jnp.float32),
        pltpu.VMEM((128,), jnp.float32),
        pltpu.VMEM((RPT,), jnp.float32),
        pltpu.VMEM((RPT // 2, 128), jnp.float32),
        pltpu.VMEM_SHARED((NPAD,), jnp.float32),
        pltpu.SemaphoreType.DMA,
    ],
)


def _agg_body(g_hbm, eidx_hbm, out_hbm,
              idxv, b0, b1, b2, b3, b4, b5, b6, b7,
              g0, g1, g2, g3, g4, g5, g6, g7,
              s0, s1, s2, s3, s4, s5, s6, s7, acc):
    bufs = [b0, b1, b2, b3, b4, b5, b6, b7]
    gs = [g0, g1, g2, g3, g4, g5, g6, g7]
    ss = [s0, s1, s2, s3, s4, s5, s6, s7]
    c = lax.axis_index("c")
    s = lax.axis_index("s")
    w = c * NS + s
    pltpu.sync_copy(eidx_hbm.at[pl.ds(w * NCA, NCA)], idxv)

    def zrow(i, carry):
        for kk in range(H // 16):
            b0[i, pl.ds(16 * kk, 16)] = jnp.zeros((16,), jnp.float32)
        return carry

    lax.fori_loop(0, KA, zrow, 0)
    r0 = s * RPA

    @pl.when(jnp.logical_and(c == 0, s < NS - 1))
    def _():
        pltpu.sync_copy(g_hbm.at[pl.ds(r0, RPA)], acc.at[pl.ds(r0, RPA)])

    @pl.when(jnp.logical_and(c == 0, s == NS - 1))
    def _():
        pltpu.sync_copy(g_hbm.at[pl.ds((NS - 1) * RPA, N - (NS - 1) * RPA)],
                        acc.at[pl.ds((NS - 1) * RPA, N - (NS - 1) * RPA)])
        for i in range(NJUNK // 128):
            pltpu.sync_copy(b0, acc.at[pl.ds(N + i * 128, 128)])
        pltpu.sync_copy(b0.at[pl.ds(0, NJUNK % 128)],
                        acc.at[pl.ds(N + (NJUNK // 128) * 128, NJUNK % 128)])

    @pl.when(c == 1)
    def _():
        for i in range(RPA // KA):
            pltpu.sync_copy(b0, acc.at[pl.ds(r0 + i * KA, KA)])

    plsc.subcore_barrier()

    def gf(j, buf, sem):
        pltpu.async_copy(g_hbm.at[idxv.at[j, 0]], buf, sem)

    def gw(j, buf, sem):
        pltpu.make_async_copy(g_hbm.at[idxv.at[j, 0]], buf, sem).wait()

    def sf(j, buf, sem):
        pltpu.async_copy(buf, acc.at[idxv.at[j, 1]], sem, add=True)

    def sw(j, buf, sem):
        pltpu.make_async_copy(buf, acc.at[idxv.at[j, 1]], sem).wait()

    gf(0, bufs[0], gs[0])
    gf(1, bufs[1], gs[1])
    gf(2, bufs[2], gs[2])
    gf(3, bufs[3], gs[3])
    gf(4, bufs[4], gs[4])
    gf(5, bufs[5], gs[5])

    def step(t, carry):
        for k in range(4):
            j = 8 * t + 2 * k
            a0, a1 = 2 * k, 2 * k + 1
            p0, p1 = (2 * k - 2) % 8, (2 * k - 1) % 8
            gw(j, bufs[a0], gs[a0])
            gw(j + 1, bufs[a1], gs[a1])
            sf(j, bufs[a0], ss[a0])
            sf(j + 1, bufs[a1], ss[a1])

            def waits(j=j, p0=p0, p1=p1):
                sw(j - 2, bufs[p0], ss[p0])
                sw(j - 1, bufs[p1], ss[p1])

            def fires(j=j, p0=p0, p1=p1):
                gf(j + 6, bufs[p0], gs[p0])
                gf(j + 7, bufs[p1], gs[p1])

            if k == 0:
                pl.when(t > 0)(waits)
                fires()
            else:
                waits()
                pl.when(t < NCA // 8 - 1)(fires)
        return carry

    lax.fori_loop(0, NCA // 8, step, 0)
    sw(NCA - 2, bufs[6], ss[6])
    sw(NCA - 1, bufs[7], ss[7])
    plsc.subcore_barrier()
    pltpu.sync_copy(acc.at[pl.ds(r0, RPA)], out_hbm.at[c, pl.ds(r0, RPA)])


_agg_kernel = pl.kernel(
    _agg_body,
    out_type=jax.ShapeDtypeStruct((NC, ACCN, H), jnp.float32),
    mesh=_mesh,
    compiler_params=pltpu.CompilerParams(use_tc_tiling_on_sc=False),
    scratch_types=(
        [pltpu.VMEM((NCA, 2, KA), jnp.int32)]
        + [pltpu.VMEM((KA, H), jnp.float32)] * 8
        + [pltpu.SemaphoreType.DMA] * 16
        + [pltpu.VMEM_SHARED((ACCN, H), jnp.float32)]
    ),
)


NPR = N // 2
BROW = NC * ACCN * H // 128
BOFF = ACCN * H // 128


def _tc1_body(xp_ref, w1bd_ref, degb_ref, g1_ref, dinv_ref):
    dsum = degb_ref[0, :NPR] + degb_ref[1, :NPR]
    dinv = lax.rsqrt(dsum + 1.0)
    h1p = jnp.dot(xp_ref[...], w1bd_ref[...],
                  preferred_element_type=jnp.float32)
    g1_ref[...] = h1p * dinv
    dinv_ref[...] = dinv


def _tc2_body(aggv_ref, dinv_ref, b1_ref, w2bd_ref, g2_ref):
    t = ((aggv_ref[0:NPR] + aggv_ref[BOFF:BOFF + NPR]) * dinv_ref[...]
         + b1_ref[...])
    t = jnp.maximum(t, 0.0)
    h2p = jnp.dot(t, w2bd_ref[...], preferred_element_type=jnp.float32)
    g2_ref[...] = h2p * dinv_ref[...]


def _tc3_body(aggv_ref, dinv_ref, b2_ref, bate_ref, bato_ref, gat_ref,
              wp_ref, wg_ref, bf1_ref, wf2_ref, bf2_ref, out_ref):
    h = ((aggv_ref[0:NPR] + aggv_ref[BOFF:BOFF + NPR]) * dinv_ref[...]
         + b2_ref[...])
    h = jnp.maximum(h, 0.0)
    gid = lax.broadcasted_iota(jnp.int32, (G, NPR), 0)
    maske = (gid == bate_ref[...]).astype(jnp.float32)
    masko = (gid == bato_ref[...]).astype(jnp.float32)
    counts = (jnp.sum(maske, axis=1, keepdims=True)
              + jnp.sum(masko, axis=1, keepdims=True))
    pe = jnp.dot(maske, h, preferred_element_type=jnp.float32)
    po = jnp.dot(masko, h, preferred_element_type=jnp.float32)
    pooled = pe[:, 0:H] + po[:, H:2 * H]
    pooled = pooled / jnp.maximum(counts, 1.0)
    z = (jnp.dot(pooled, wp_ref[...], preferred_element_type=jnp.float32)
         + jnp.dot(gat_ref[...], wg_ref[...], preferred_element_type=jnp.float32)
         + bf1_ref[...])
    z = jnp.maximum(z, 0.0)
    out_ref[...] = (jnp.dot(z, wf2_ref[...], preferred_element_type=jnp.float32)
                    + bf2_ref[...])


def _tc_call(body, out_shape, *args):
    return pl.pallas_call(body, out_shape=out_shape)(*args)


def kernel(x, edge_index, batch, global_attr, W1, b1, W2, b2,
           Wfc1, bfc1, Wfc2, bfc2):
    pad_i = jnp.arange(EP - E, dtype=jnp.int32)
    pad = jnp.stack([pad_i * 13 % N, N + pad_i % NJUNK])
    ei = jnp.concatenate([edge_index, pad], axis=1)
    eidx = ei.reshape(2, NBLK, KA).transpose(1, 0, 2)

    degb = _deg_kernel(eidx)

    xp = x.reshape(NPR, 2 * D)
    zw = jnp.zeros((D, H), jnp.float32)
    w1bd = jnp.block([[W1, zw], [zw, W1]])
    g1p, dinvp = _tc_call(
        _tc1_body,
        (jax.ShapeDtypeStruct((NPR, 2 * H), jnp.float32),
         jax.ShapeDtypeStruct((NPR, 2 * H), jnp.float32)),
        xp, w1bd, degb)

    zb = jnp.zeros((H, H), jnp.float32)
    w2bd = jnp.block([[W2, zb], [zb, W2]])
    b1p = jnp.concatenate([b1, b1]).reshape(1, 2 * H)
    b2p = jnp.concatenate([b2, b2]).reshape(1, 2 * H)
    bpair = batch.reshape(NPR, 2)
    bate = bpair[:, 0].reshape(1, NPR)
    bato = bpair[:, 1].reshape(1, NPR)

    agg1 = _agg_kernel(g1p.reshape(N, H), eidx)
    g2p = _tc_call(
        _tc2_body, jax.ShapeDtypeStruct((NPR, 2 * H), jnp.float32),
        agg1.reshape(BROW, 128), dinvp, b1p, w2bd)

    agg2 = _agg_kernel(g2p.reshape(N, H), eidx)
    out = _tc_call(
        _tc3_body, jax.ShapeDtypeStruct((G, 1), jnp.float32),
        agg2.reshape(BROW, 128), dinvp, b2p, bate, bato, global_attr,
        Wfc1[:H], Wfc1[H:], bfc1.reshape(1, 64), Wfc2, bfc2.reshape(1, 1))
    return out.reshape(G)


# speedup vs baseline: 3.5726x; 1.0464x over previous
import jax
import jax.numpy as jnp
from jax import lax
from jax.experimental import pallas as pl
from jax.experimental.pallas import tpu as pltpu
from jax.experimental.pallas import tpu_sc as plsc

N = 10000
E = 320000
D = 128
H = 64
G = 64
GF = 16

NC = 2
NS = 16
NW = NC * NS
KA = 128
NCA = 80
EP = NW * NCA * KA
NBLK = EP // KA
ACCN = 10240
RPA = ACCN // NS
NJUNK = ACCN - N
NPAD = 10240
RPT = NPAD // NS

_mesh = plsc.VectorSubcoreMesh(
    core_axis_name="c", subcore_axis_name="s", num_cores=NC, num_subcores=NS)


NBLK0 = E // KA
DB = NBLK0 // NW
DREM = NBLK0 % NW


def _deg_body(eidx_hbm, out_hbm, idxv, ones_v, zbuf, dld, dbb, acc, sem):
    c = lax.axis_index("c")
    s = lax.axis_index("s")
    w = c * NS + s
    nb = DB + jnp.where(w < DREM, 1, 0)
    b0 = DB * w + jnp.minimum(w, DREM)

    @pl.when(w < DREM)
    def _():
        pltpu.sync_copy(eidx_hbm.at[pl.ds(b0, DB + 1)], idxv)

    @pl.when(w >= DREM)
    def _():
        pltpu.sync_copy(eidx_hbm.at[pl.ds(b0, DB)], idxv.at[pl.ds(0, DB)])

    for i in range(8):
        ones_v[pl.ds(i * 16, 16)] = jnp.ones((16,), jnp.float32)
        zbuf[pl.ds(i * 16, 16)] = jnp.zeros((16,), jnp.float32)
    r0 = s * RPT
    for i in range(RPT // 128):
        pltpu.sync_copy(zbuf, acc.at[pl.ds(r0 + i * 128, 128)])
    plsc.subcore_barrier()

    def fire(j, carry):
        pltpu.async_copy(ones_v, acc.at[idxv.at[j, 1]], sem, add=True)
        return carry

    lax.fori_loop(0, nb, fire, 0)

    def drain(j, carry):
        pltpu.make_async_copy(ones_v, acc.at[idxv.at[j, 1]], sem).wait()
        return carry

    lax.fori_loop(0, nb, drain, 0)
    plsc.subcore_barrier()
    pltpu.sync_copy(acc.at[pl.ds(r0, RPT)], dld)

    def bcast(i, carry):
        v0 = plsc.load_gather(dld, [jnp.full((16,), 2 * i, jnp.int32)])
        v1 = plsc.load_gather(dld, [jnp.full((16,), 2 * i + 1, jnp.int32)])
        for k in range(4):
            dbb[i, pl.ds(16 * k, 16)] = v0
        for k in range(4, 8):
            dbb[i, pl.ds(16 * k, 16)] = v1
        return carry

    lax.fori_loop(0, RPT // 2, bcast, 0)
    pltpu.sync_copy(dbb, out_hbm.at[c, pl.ds(s * (RPT // 2), RPT // 2)])


_deg_kernel = pl.kernel(
    _deg_body,
    out_type=jax.ShapeDtypeStruct((NC, NPAD // 2, 128), jnp.float32),
    mesh=_mesh,
    compiler_params=pltpu.CompilerParams(needs_layout_passes=False),
    scratch_types=[
        pltpu.VMEM((DB + 1, 2, KA), jnp.int32),
        pltpu.VMEM((KA,), jnp.float32),
        pltpu.VMEM((128,), jnp.float32),
        pltpu.VMEM((RPT,), jnp.float32),
        pltpu.VMEM((RPT // 2, 128), jnp.float32),
        pltpu.VMEM_SHARED((NPAD,), jnp.float32),
        pltpu.SemaphoreType.DMA,
    ],
)


def _agg_body(g_hbm, eidx_hbm, out_hbm,
              idxv, b0, b1, b2, b3, b4, b5, b6, b7,
              g0, g1, g2, g3, g4, g5, g6, g7,
              s0, s1, s2, s3, s4, s5, s6, s7, acc):
    bufs = [b0, b1, b2, b3, b4, b5, b6, b7]
    gs = [g0, g1, g2, g3, g4, g5, g6, g7]
    ss = [s0, s1, s2, s3, s4, s5, s6, s7]
    c = lax.axis_index("c")
    s = lax.axis_index("s")
    w = c * NS + s
    pltpu.sync_copy(eidx_hbm.at[pl.ds(w * NCA, NCA)], idxv)

    def zrow(i, carry):
        for kk in range(H // 16):
            b0[i, pl.ds(16 * kk, 16)] = jnp.zeros((16,), jnp.float32)
        return carry

    lax.fori_loop(0, KA, zrow, 0)
    r0 = s * RPA

    @pl.when(jnp.logical_and(c == 0, s < NS - 1))
    def _():
        pltpu.sync_copy(g_hbm.at[pl.ds(r0, RPA)], acc.at[pl.ds(r0, RPA)])

    @pl.when(jnp.logical_and(c == 0, s == NS - 1))
    def _():
        pltpu.sync_copy(g_hbm.at[pl.ds((NS - 1) * RPA, N - (NS - 1) * RPA)],
                        acc.at[pl.ds((NS - 1) * RPA, N - (NS - 1) * RPA)])
        for i in range(NJUNK // 128):
            pltpu.sync_copy(b0, acc.at[pl.ds(N + i * 128, 128)])
        pltpu.sync_copy(b0.at[pl.ds(0, NJUNK % 128)],
                        acc.at[pl.ds(N + (NJUNK // 128) * 128, NJUNK % 128)])

    @pl.when(c == 1)
    def _():
        for i in range(RPA // KA):
            pltpu.sync_copy(b0, acc.at[pl.ds(r0 + i * KA, KA)])

    plsc.subcore_barrier()

    def gf(j, buf, sem):
        pltpu.async_copy(g_hbm.at[idxv.at[j, 0]], buf, sem)

    def gw(j, buf, sem):
        pltpu.make_async_copy(g_hbm.at[idxv.at[j, 0]], buf, sem).wait()

    def sf(j, buf, sem):
        pltpu.async_copy(buf, acc.at[idxv.at[j, 1]], sem, add=True)

    def sw(j, buf, sem):
        pltpu.make_async_copy(buf, acc.at[idxv.at[j, 1]], sem).wait()

    gf(0, bufs[0], gs[0])
    gf(1, bufs[1], gs[1])
    gf(2, bufs[2], gs[2])
    gf(3, bufs[3], gs[3])
    gf(4, bufs[4], gs[4])
    gf(5, bufs[5], gs[5])

    def step(t, carry):
        for k in range(4):
            j = 8 * t + 2 * k
            a0, a1 = 2 * k, 2 * k + 1
            p0, p1 = (2 * k - 2) % 8, (2 * k - 1) % 8
            gw(j, bufs[a0], gs[a0])
            gw(j + 1, bufs[a1], gs[a1])
            sf(j, bufs[a0], ss[a0])
            sf(j + 1, bufs[a1], ss[a1])

            def waits(j=j, p0=p0, p1=p1):
                sw(j - 2, bufs[p0], ss[p0])
                sw(j - 1, bufs[p1], ss[p1])

            def fires(j=j, p0=p0, p1=p1):
                gf(j + 6, bufs[p0], gs[p0])
                gf(j + 7, bufs[p1], gs[p1])

            if k == 0:
                pl.when(t > 0)(waits)
                fires()
            else:
                waits()
                pl.when(t < NCA // 8 - 1)(fires)
        return carry

    lax.fori_loop(0, NCA // 8, step, 0)
    sw(NCA - 2, bufs[6], ss[6])
    sw(NCA - 1, bufs[7], ss[7])
    plsc.subcore_barrier()
    pltpu.sync_copy(acc.at[pl.ds(r0, RPA)], out_hbm.at[c, pl.ds(r0, RPA)])


_agg_kernel = pl.kernel(
    _agg_body,
    out_type=jax.ShapeDtypeStruct((NC, ACCN, H), jnp.float32),
    mesh=_mesh,
    compiler_params=pltpu.CompilerParams(use_tc_tiling_on_sc=False),
    scratch_types=(
        [pltpu.VMEM((NCA, 2, KA), jnp.int32)]
        + [pltpu.VMEM((KA, H), jnp.float32)] * 8
        + [pltpu.SemaphoreType.DMA] * 16
        + [pltpu.VMEM_SHARED((ACCN, H), jnp.float32)]
    ),
)


NPR = N // 2
BROW = NC * ACCN * H // 128
BOFF = ACCN * H // 128


def _tc1_body(xp_ref, w1bd_ref, degb_ref, g1_ref, dinv_ref):
    dsum = degb_ref[0, :NPR] + degb_ref[1, :NPR]
    dinv = lax.rsqrt(dsum + 1.0)
    h1p = jnp.dot(xp_ref[...], w1bd_ref[...],
                  preferred_element_type=jnp.float32)
    g1_ref[...] = h1p * dinv
    dinv_ref[...] = dinv


def _tc2_body(aggv_ref, dinv_ref, b1_ref, w2bd_ref, g2_ref):
    t = ((aggv_ref[0:NPR] + aggv_ref[BOFF:BOFF + NPR]) * dinv_ref[...]
         + b1_ref[...])
    t = jnp.maximum(t, 0.0)
    h2p = jnp.dot(t, w2bd_ref[...], preferred_element_type=jnp.float32)
    g2_ref[...] = h2p * dinv_ref[...]


def _tc3_body(aggv_ref, dinv_ref, b2_ref, bate_ref, bato_ref, gat_ref,
              wp_ref, wg_ref, bf1_ref, wf2_ref, bf2_ref, out_ref):
    h = ((aggv_ref[0:NPR] + aggv_ref[BOFF:BOFF + NPR]) * dinv_ref[...]
         + b2_ref[...])
    h = jnp.maximum(h, 0.0)
    gid = lax.broadcasted_iota(jnp.int32, (G, NPR), 0)
    maske = (gid == bate_ref[...]).astype(jnp.float32)
    masko = (gid == bato_ref[...]).astype(jnp.float32)
    counts = (jnp.sum(maske, axis=1, keepdims=True)
              + jnp.sum(masko, axis=1, keepdims=True))
    pe = jnp.dot(maske, h, preferred_element_type=jnp.float32)
    po = jnp.dot(masko, h, preferred_element_type=jnp.float32)
    pooled = pe[:, 0:H] + po[:, H:2 * H]
    pooled = pooled / jnp.maximum(counts, 1.0)
    z = (jnp.dot(pooled, wp_ref[...], preferred_element_type=jnp.float32)
         + jnp.dot(gat_ref[...], wg_ref[...], preferred_element_type=jnp.float32)
         + bf1_ref[...])
    z = jnp.maximum(z, 0.0)
    out_ref[...] = (jnp.dot(z, wf2_ref[...], preferred_element_type=jnp.float32)
                    + bf2_ref[...])


def _tc_call(body, out_shape, *args):
    return pl.pallas_call(body, out_shape=out_shape)(*args)


def kernel(x, edge_index, batch, global_attr, W1, b1, W2, b2,
           Wfc1, bfc1, Wfc2, bfc2):
    pad_i = jnp.arange(EP - E, dtype=jnp.int32)
    pad = jnp.stack([pad_i * 13 % N, N + pad_i % NJUNK])
    ei = jnp.concatenate([edge_index, pad], axis=1)
    eidx = ei.reshape(2, NBLK, KA).transpose(1, 0, 2)

    eidx0 = edge_index.reshape(2, NBLK0, KA).transpose(1, 0, 2)
    degb = _deg_kernel(eidx0)

    xp = x.reshape(NPR, 2 * D)
    zw = jnp.zeros((D, H), jnp.float32)
    w1bd = jnp.block([[W1, zw], [zw, W1]])
    g1p, dinvp = _tc_call(
        _tc1_body,
        (jax.ShapeDtypeStruct((NPR, 2 * H), jnp.float32),
         jax.ShapeDtypeStruct((NPR, 2 * H), jnp.float32)),
        xp, w1bd, degb)

    zb = jnp.zeros((H, H), jnp.float32)
    w2bd = jnp.block([[W2, zb], [zb, W2]])
    b1p = jnp.concatenate([b1, b1]).reshape(1, 2 * H)
    b2p = jnp.concatenate([b2, b2]).reshape(1, 2 * H)
    bpair = batch.reshape(NPR, 2)
    bate = bpair[:, 0].reshape(1, NPR)
    bato = bpair[:, 1].reshape(1, NPR)

    agg1 = _agg_kernel(g1p.reshape(N, H), eidx)
    g2p = _tc_call(
        _tc2_body, jax.ShapeDtypeStruct((NPR, 2 * H), jnp.float32),
        agg1.reshape(BROW, 128), dinvp, b1p, w2bd)

    agg2 = _agg_kernel(g2p.reshape(N, H), eidx)
    out = _tc_call(
        _tc3_body, jax.ShapeDtypeStruct((G, 1), jnp.float32),
        agg2.reshape(BROW, 128), dinvp, b2p, bate, bato, global_attr,
        Wfc1[:H], Wfc1[H:], bfc1.reshape(1, 64), Wfc2, bfc2.reshape(1, 1))
    return out.reshape(G)
